# baseline (speedup 1.0000x reference)
import functools

import jax
import jax.numpy as jnp
from jax import lax
from jax.experimental import pallas as pl
from jax.experimental.pallas import tpu as pltpu
from jax.experimental.pallas import tpu_sc as plsc

_N = 10000
_D = 128
_NC = 2
_NS = 16
_NW = _NC * _NS
_K = 64
_NPAD = 10240
_RS = _NPAD // _NS
_R = 1000

_HIGH = jax.lax.Precision.HIGHEST


_B = 32
_NBUF = 4
_DEPTH = 3


def _sc_body(with_deg, n_chunks, ew, *refs):
    if with_deg:
        (y, srcr, dstr, aggout, degout, src_blk, dst_blk) = refs[:7]
        bufs = refs[7:7 + _NBUF]
        (ones, degtmp, aggsh, degsh) = refs[7 + _NBUF:11 + _NBUF]
        sems = refs[11 + _NBUF:]
    else:
        (y, srcr, dstr, aggout, src_blk, dst_blk) = refs[:6]
        bufs = refs[6:6 + _NBUF]
        aggsh = refs[6 + _NBUF]
        sems = refs[7 + _NBUF:]
    gsems = sems[:_NBUF]
    ssems = sems[_NBUF:2 * _NBUF]
    dsems = sems[2 * _NBUF:]
    rows = bufs[0]

    c = lax.axis_index("c")
    s = lax.axis_index("s")
    w = c * _NS + s

    zeros16 = jnp.zeros((16,), jnp.float32)

    def zrow(i, carry):
        for j in range(_D // 16):
            rows[i, pl.ds(j * 16, 16)] = zeros16
        return carry

    lax.fori_loop(0, _K, zrow, 0)
    base = s * _RS
    for j in range(_RS // _K):
        pltpu.sync_copy(rows, aggsh.at[pl.ds(base + j * _K, _K)])

    if with_deg:
        def zdeg(i, carry):
            degtmp[pl.ds(i * 16, 16)] = zeros16
            return carry

        lax.fori_loop(0, _RS // 16, zdeg, 0)
        pltpu.sync_copy(degtmp, degsh.at[pl.ds(base, _RS)])

        def fill_ones(i, carry):
            ones[pl.ds(i * 16, 16)] = jnp.ones((16,), jnp.float32)
            return carry

        lax.fori_loop(0, _K // 16, fill_ones, 0)

    plsc.subcore_barrier()

    wrow = w * (ew // _K)

    def blk(b, carry):
        r0 = wrow + b * _B
        pltpu.sync_copy(srcr.at[pl.ds(r0, _B)], src_blk)
        pltpu.sync_copy(dstr.at[pl.ds(r0, _B)], dst_blk)
        gd = [None] * _B
        sd = [None] * _B
        dd = [None] * _B

        def scatter(i):
            gd[i].wait()
            sd[i] = pltpu.async_copy(bufs[i % _NBUF],
                                     aggsh.at[dst_blk.at[i]],
                                     ssems[i % _NBUF], add=True)
            if with_deg:
                dd[i] = pltpu.async_copy(ones, degsh.at[dst_blk.at[i]],
                                         dsems[i % _NBUF], add=True)

        for j in range(_B):
            if j >= _NBUF:
                sd[j - _NBUF].wait()
                if with_deg:
                    dd[j - _NBUF].wait()
            gd[j] = pltpu.async_copy(y.at[src_blk.at[j]],
                                     bufs[j % _NBUF], gsems[j % _NBUF])
            if j >= _DEPTH:
                scatter(j - _DEPTH)
        for i in range(_B - _DEPTH, _B):
            scatter(i)
        for i in range(_B - _NBUF, _B):
            sd[i].wait()
            if with_deg:
                dd[i].wait()
        return carry

    lax.fori_loop(0, n_chunks // _B, blk, 0)

    plsc.subcore_barrier()

    pltpu.sync_copy(aggsh.at[pl.ds(base, _RS)],
                    aggout.at[c, pl.ds(base, _RS)])
    if with_deg:
        pltpu.sync_copy(degsh.at[pl.ds(base, _RS)],
                        degout.at[c, pl.ds(base, _RS)])


@functools.lru_cache(maxsize=None)
def _make_sc_segsum(e_pad, with_deg):
    ew = e_pad // _NW
    n_chunks = ew // _K
    mesh = plsc.VectorSubcoreMesh(core_axis_name="c", subcore_axis_name="s")
    out_type = [jax.ShapeDtypeStruct((_NC, _NPAD, _D), jnp.float32)]
    scratch = [
        pltpu.VMEM((_B, _K), jnp.int32),
        pltpu.VMEM((_B, _K), jnp.int32),
    ]
    scratch += [pltpu.VMEM((_K, _D), jnp.float32)] * _NBUF
    assert 16 * (_NBUF * _K * _D + 2 * _B * _K + _K + _RS) \
        + _NPAD * _D + _NPAD <= 2097151
    if with_deg:
        out_type.append(jax.ShapeDtypeStruct((_NC, _NPAD), jnp.float32))
        scratch += [
            pltpu.VMEM((_K,), jnp.float32),
            pltpu.VMEM((_RS,), jnp.float32),
        ]
    scratch.append(pltpu.VMEM_SHARED((_NPAD, _D), jnp.float32))
    if with_deg:
        scratch.append(pltpu.VMEM_SHARED((_NPAD,), jnp.float32))
    scratch += [pltpu.SemaphoreType.DMA] * (_NBUF * (3 if with_deg else 2))
    return pl.kernel(
        functools.partial(_sc_body, with_deg, n_chunks, ew),
        out_type=tuple(out_type),
        mesh=mesh,
        scratch_types=tuple(scratch),
        name=f"sage_sc_segsum{'_deg' if with_deg else ''}",
    )


def _mm_body(x_r, wl_r, y_r):
    y_r[...] = jnp.dot(x_r[...], wl_r[...], precision=_HIGH,
                       preferred_element_type=jnp.float32)


def _combine_mm_body(p_r, d0_r, d1_r, hp_r, wr_r, bl_r, wl_r, h_r, y_r):
    deg = jnp.maximum(d0_r[...] + d1_r[...], 1.0)
    z = jnp.dot(hp_r[...], wr_r[...], precision=_HIGH,
                preferred_element_type=jnp.float32) + bl_r[...]
    h = jnp.maximum((p_r[0] + p_r[1]) / deg + z, 0.0)
    h_r[...] = h
    y_r[...] = jnp.dot(h, wl_r[...], precision=_HIGH,
                       preferred_element_type=jnp.float32)


def _combine_body(p_r, d0_r, d1_r, hp_r, wr_r, bl_r, h_r):
    deg = jnp.maximum(d0_r[...] + d1_r[...], 1.0)
    z = jnp.dot(hp_r[...], wr_r[...], precision=_HIGH,
                preferred_element_type=jnp.float32) + bl_r[...]
    h_r[...] = jnp.maximum((p_r[0] + p_r[1]) / deg + z, 0.0)


_row_spec = pl.BlockSpec((_R, _D), lambda i: (i, 0))
_w_spec = pl.BlockSpec((_D, _D), lambda i: (0, 0))
_b_spec = pl.BlockSpec((1, _D), lambda i: (0, 0))
_deg_spec = pl.BlockSpec((_R, 1), lambda i: (i, 0))
_p_spec = pl.BlockSpec((_NC, _R, _D), lambda i: (0, i, 0))
_nd = jax.ShapeDtypeStruct((_N, _D), jnp.float32)

_mm = pl.pallas_call(
    _mm_body,
    grid=(_N // _R,),
    in_specs=[_row_spec, _w_spec],
    out_specs=_row_spec,
    out_shape=_nd,
)

_combine_mm = pl.pallas_call(
    _combine_mm_body,
    grid=(_N // _R,),
    in_specs=[_p_spec, _deg_spec, _deg_spec, _row_spec,
              _w_spec, _b_spec, _w_spec],
    out_specs=[_row_spec, _row_spec],
    out_shape=[_nd, _nd],
)

_combine = pl.pallas_call(
    _combine_body,
    grid=(_N // _R,),
    in_specs=[_p_spec, _deg_spec, _deg_spec, _row_spec, _w_spec, _b_spec],
    out_specs=_row_spec,
    out_shape=_nd,
)


def kernel(x, edge_index, Wl0, bl0, Wr0, Wl1, bl1, Wr1, Wl2, bl2, Wr2):
    src = edge_index[0]
    dst = edge_index[1]
    e = src.shape[0]
    span = _NW * _K * _B
    e_pad = -(-e // span) * span
    pad = e_pad - e
    if pad:
        ar = jnp.arange(pad, dtype=jnp.int32)
        src = jnp.concatenate([src, ar % _N])
        dst = jnp.concatenate([dst, _N + ar % (_NPAD - _N)])
    src = src.reshape(-1, _K)
    dst = dst.reshape(-1, _K)

    sc_deg = _make_sc_segsum(e_pad, True)
    sc = _make_sc_segsum(e_pad, False)

    y0 = _mm(x, Wl0)
    p, pdeg = sc_deg(y0, src, dst)
    d0 = pdeg[0, :_N].reshape(_N, 1)
    d1 = pdeg[1, :_N].reshape(_N, 1)
    h1, y1 = _combine_mm(p, d0, d1, x, Wr0, bl0.reshape(1, _D), Wl1)
    (q,) = sc(y1, src, dst)
    h2, y2 = _combine_mm(q, d0, d1, h1, Wr1, bl1.reshape(1, _D), Wl2)
    (r,) = sc(y2, src, dst)
    h3 = _combine(r, d0, d1, h2, Wr2, bl2.reshape(1, _D))
    return (h1, h2, h3)


# speedup vs baseline: 1.0990x; 1.0990x over previous
import functools

import jax
import jax.numpy as jnp
from jax import lax
from jax.experimental import pallas as pl
from jax.experimental.pallas import tpu as pltpu
from jax.experimental.pallas import tpu_sc as plsc

_N = 10000
_D = 128
_NC = 2
_NS = 16
_NW = _NC * _NS
_K = 64
_NPAD = 10240
_RS = _NPAD // _NS
_R = 1000

_HIGH = jax.lax.Precision.DEFAULT


_B = 32
_NBUF = 4
_DEPTH = 3


def _sc_body(with_deg, n_chunks, ew, *refs):
    if with_deg:
        (y, srcr, dstr, aggout, degout, src_blk, dst_blk) = refs[:7]
        bufs = refs[7:7 + _NBUF]
        (ones, degtmp, aggsh, degsh) = refs[7 + _NBUF:11 + _NBUF]
        sems = refs[11 + _NBUF:]
    else:
        (y, srcr, dstr, aggout, src_blk, dst_blk) = refs[:6]
        bufs = refs[6:6 + _NBUF]
        aggsh = refs[6 + _NBUF]
        sems = refs[7 + _NBUF:]
    gsems = sems[:_NBUF]
    ssems = sems[_NBUF:2 * _NBUF]
    dsems = sems[2 * _NBUF:]
    rows = bufs[0]

    c = lax.axis_index("c")
    s = lax.axis_index("s")
    w = c * _NS + s

    zeros16 = jnp.zeros((16,), jnp.float32)

    def zrow(i, carry):
        for j in range(_D // 16):
            rows[i, pl.ds(j * 16, 16)] = zeros16
        return carry

    lax.fori_loop(0, _K, zrow, 0)
    base = s * _RS
    for j in range(_RS // _K):
        pltpu.sync_copy(rows, aggsh.at[pl.ds(base + j * _K, _K)])

    if with_deg:
        def zdeg(i, carry):
            degtmp[pl.ds(i * 16, 16)] = zeros16
            return carry

        lax.fori_loop(0, _RS // 16, zdeg, 0)
        pltpu.sync_copy(degtmp, degsh.at[pl.ds(base, _RS)])

        def fill_ones(i, carry):
            ones[pl.ds(i * 16, 16)] = jnp.ones((16,), jnp.float32)
            return carry

        lax.fori_loop(0, _K // 16, fill_ones, 0)

    plsc.subcore_barrier()

    wrow = w * (ew // _K)

    def blk(b, carry):
        r0 = wrow + b * _B
        pltpu.sync_copy(srcr.at[pl.ds(r0, _B)], src_blk)
        pltpu.sync_copy(dstr.at[pl.ds(r0, _B)], dst_blk)
        gd = [None] * _B
        sd = [None] * _B
        dd = [None] * _B

        def scatter(i):
            gd[i].wait()
            sd[i] = pltpu.async_copy(bufs[i % _NBUF],
                                     aggsh.at[dst_blk.at[i]],
                                     ssems[i % _NBUF], add=True)
            if with_deg:
                dd[i] = pltpu.async_copy(ones, degsh.at[dst_blk.at[i]],
                                         dsems[i % _NBUF], add=True)

        for j in range(_B):
            if j >= _NBUF:
                sd[j - _NBUF].wait()
                if with_deg:
                    dd[j - _NBUF].wait()
            gd[j] = pltpu.async_copy(y.at[src_blk.at[j]],
                                     bufs[j % _NBUF], gsems[j % _NBUF])
            if j >= _DEPTH:
                scatter(j - _DEPTH)
        for i in range(_B - _DEPTH, _B):
            scatter(i)
        for i in range(_B - _NBUF, _B):
            sd[i].wait()
            if with_deg:
                dd[i].wait()
        return carry

    lax.fori_loop(0, n_chunks // _B, blk, 0)

    plsc.subcore_barrier()

    pltpu.sync_copy(aggsh.at[pl.ds(base, _RS)],
                    aggout.at[c, pl.ds(base, _RS)])
    if with_deg:
        pltpu.sync_copy(degsh.at[pl.ds(base, _RS)],
                        degout.at[c, pl.ds(base, _RS)])


@functools.lru_cache(maxsize=None)
def _make_sc_segsum(e_pad, with_deg):
    ew = e_pad // _NW
    n_chunks = ew // _K
    mesh = plsc.VectorSubcoreMesh(core_axis_name="c", subcore_axis_name="s")
    out_type = [jax.ShapeDtypeStruct((_NC, _NPAD, _D), jnp.float32)]
    scratch = [
        pltpu.VMEM((_B, _K), jnp.int32),
        pltpu.VMEM((_B, _K), jnp.int32),
    ]
    scratch += [pltpu.VMEM((_K, _D), jnp.float32)] * _NBUF
    assert 16 * (_NBUF * _K * _D + 2 * _B * _K + _K + _RS) \
        + _NPAD * _D + _NPAD <= 2097151
    if with_deg:
        out_type.append(jax.ShapeDtypeStruct((_NC, _NPAD), jnp.float32))
        scratch += [
            pltpu.VMEM((_K,), jnp.float32),
            pltpu.VMEM((_RS,), jnp.float32),
        ]
    scratch.append(pltpu.VMEM_SHARED((_NPAD, _D), jnp.float32))
    if with_deg:
        scratch.append(pltpu.VMEM_SHARED((_NPAD,), jnp.float32))
    scratch += [pltpu.SemaphoreType.DMA] * (_NBUF * (3 if with_deg else 2))
    return pl.kernel(
        functools.partial(_sc_body, with_deg, n_chunks, ew),
        out_type=tuple(out_type),
        mesh=mesh,
        scratch_types=tuple(scratch),
        name=f"sage_sc_segsum{'_deg' if with_deg else ''}",
    )


def _mm_body(x_r, wl_r, wr_r, bl_r, y_r, z_r):
    h = x_r[...]
    y_r[...] = jnp.dot(h, wl_r[...], precision=_HIGH,
                       preferred_element_type=jnp.float32)
    z_r[...] = jnp.dot(h, wr_r[...], precision=_HIGH,
                       preferred_element_type=jnp.float32) + bl_r[...]


def _combine_mm_body(p_r, d0_r, d1_r, z_r, wl_r, wr_r, bl_r,
                     h_r, y_r, z2_r):
    deg = jnp.maximum(d0_r[...] + d1_r[...], 1.0)
    h = jnp.maximum((p_r[0] + p_r[1]) / deg + z_r[...], 0.0)
    h_r[...] = h
    y_r[...] = jnp.dot(h, wl_r[...], precision=_HIGH,
                       preferred_element_type=jnp.float32)
    z2_r[...] = jnp.dot(h, wr_r[...], precision=_HIGH,
                        preferred_element_type=jnp.float32) + bl_r[...]


def _combine_body(p_r, d0_r, d1_r, z_r, h_r):
    deg = jnp.maximum(d0_r[...] + d1_r[...], 1.0)
    h_r[...] = jnp.maximum((p_r[0] + p_r[1]) / deg + z_r[...], 0.0)


_row_spec = pl.BlockSpec((_R, _D), lambda i: (i, 0))
_w_spec = pl.BlockSpec((_D, _D), lambda i: (0, 0))
_b_spec = pl.BlockSpec((1, _D), lambda i: (0, 0))
_deg_spec = pl.BlockSpec((_R, 1), lambda i: (i, 0))
_p_spec = pl.BlockSpec((_NC, _R, _D), lambda i: (0, i, 0))
_nd = jax.ShapeDtypeStruct((_N, _D), jnp.float32)

_mm = pl.pallas_call(
    _mm_body,
    grid=(_N // _R,),
    in_specs=[_row_spec, _w_spec, _w_spec, _b_spec],
    out_specs=[_row_spec, _row_spec],
    out_shape=[_nd, _nd],
)

_combine_mm = pl.pallas_call(
    _combine_mm_body,
    grid=(_N // _R,),
    in_specs=[_p_spec, _deg_spec, _deg_spec, _row_spec,
              _w_spec, _w_spec, _b_spec],
    out_specs=[_row_spec, _row_spec, _row_spec],
    out_shape=[_nd, _nd, _nd],
)

_combine = pl.pallas_call(
    _combine_body,
    grid=(_N // _R,),
    in_specs=[_p_spec, _deg_spec, _deg_spec, _row_spec],
    out_specs=_row_spec,
    out_shape=_nd,
)


def kernel(x, edge_index, Wl0, bl0, Wr0, Wl1, bl1, Wr1, Wl2, bl2, Wr2):
    src = edge_index[0]
    dst = edge_index[1]
    e = src.shape[0]
    span = _NW * _K * _B
    e_pad = -(-e // span) * span
    pad = e_pad - e
    if pad:
        ar = jnp.arange(pad, dtype=jnp.int32)
        src = jnp.concatenate([src, ar % _N])
        dst = jnp.concatenate([dst, _N + ar % (_NPAD - _N)])
    src = src.reshape(-1, _K)
    dst = dst.reshape(-1, _K)

    sc_deg = _make_sc_segsum(e_pad, True)
    sc = _make_sc_segsum(e_pad, False)

    y0, z0 = _mm(x, Wl0, Wr0, bl0.reshape(1, _D))
    p, pdeg = sc_deg(y0, src, dst)
    d0 = pdeg[0, :_N].reshape(_N, 1)
    d1 = pdeg[1, :_N].reshape(_N, 1)
    h1, y1, z1 = _combine_mm(p, d0, d1, z0, Wl1, Wr1, bl1.reshape(1, _D))
    (q,) = sc(y1, src, dst)
    h2, y2, z2 = _combine_mm(q, d0, d1, z1, Wl2, Wr2, bl2.reshape(1, _D))
    (r,) = sc(y2, src, dst)
    h3 = _combine(r, d0, d1, z2)
    return (h1, h2, h3)


# speedup vs baseline: 1.1225x; 1.0214x over previous
import functools

import jax
import jax.numpy as jnp
from jax import lax
from jax.experimental import pallas as pl
from jax.experimental.pallas import tpu as pltpu
from jax.experimental.pallas import tpu_sc as plsc

_N = 10000
_D = 128
_NC = 2
_NS = 16
_NW = _NC * _NS
_K = 64
_NPAD = 10240
_RS = _NPAD // _NS
_R = 2000

_HIGH = jax.lax.Precision.DEFAULT


_B = 32
_NBUF = 4
_DEPTH = 3


def _sc_body(with_deg, n_chunks, ew, *refs):
    if with_deg:
        (y, srcr, dstr, aggout, degout, src_blk, dst_blk) = refs[:7]
        bufs = refs[7:7 + _NBUF]
        (ones, degtmp, aggsh, degsh) = refs[7 + _NBUF:11 + _NBUF]
        sems = refs[11 + _NBUF:]
    else:
        (y, srcr, dstr, aggout, src_blk, dst_blk) = refs[:6]
        bufs = refs[6:6 + _NBUF]
        aggsh = refs[6 + _NBUF]
        sems = refs[7 + _NBUF:]
    gsems = sems[:_NBUF]
    ssems = sems[_NBUF:2 * _NBUF]
    dsems = sems[2 * _NBUF:]
    rows = bufs[0]

    c = lax.axis_index("c")
    s = lax.axis_index("s")
    w = c * _NS + s

    zeros16 = jnp.zeros((16,), jnp.float32)

    def zrow(i, carry):
        for j in range(_D // 16):
            rows[i, pl.ds(j * 16, 16)] = zeros16
        return carry

    lax.fori_loop(0, _K, zrow, 0)
    base = s * _RS
    for j in range(_RS // _K):
        pltpu.sync_copy(rows, aggsh.at[pl.ds(base + j * _K, _K)])

    if with_deg:
        def zdeg(i, carry):
            degtmp[pl.ds(i * 16, 16)] = zeros16
            return carry

        lax.fori_loop(0, _RS // 16, zdeg, 0)
        pltpu.sync_copy(degtmp, degsh.at[pl.ds(base, _RS)])

        def fill_ones(i, carry):
            ones[pl.ds(i * 16, 16)] = jnp.ones((16,), jnp.float32)
            return carry

        lax.fori_loop(0, _K // 16, fill_ones, 0)

    plsc.subcore_barrier()

    wrow = w * (ew // _K)

    def blk(b, carry):
        r0 = wrow + b * _B
        pltpu.sync_copy(srcr.at[pl.ds(r0, _B)], src_blk)
        pltpu.sync_copy(dstr.at[pl.ds(r0, _B)], dst_blk)
        gd = [None] * _B
        sd = [None] * _B
        dd = [None] * _B

        def scatter(i):
            gd[i].wait()
            sd[i] = pltpu.async_copy(bufs[i % _NBUF],
                                     aggsh.at[dst_blk.at[i]],
                                     ssems[i % _NBUF], add=True)
            if with_deg:
                dd[i] = pltpu.async_copy(ones, degsh.at[dst_blk.at[i]],
                                         dsems[i % _NBUF], add=True)

        for j in range(_B):
            if j >= _NBUF:
                sd[j - _NBUF].wait()
                if with_deg:
                    dd[j - _NBUF].wait()
            gd[j] = pltpu.async_copy(y.at[src_blk.at[j]],
                                     bufs[j % _NBUF], gsems[j % _NBUF])
            if j >= _DEPTH:
                scatter(j - _DEPTH)
        for i in range(_B - _DEPTH, _B):
            scatter(i)
        for i in range(_B - _NBUF, _B):
            sd[i].wait()
            if with_deg:
                dd[i].wait()
        return carry

    lax.fori_loop(0, n_chunks // _B, blk, 0)

    plsc.subcore_barrier()

    pltpu.sync_copy(aggsh.at[pl.ds(base, _RS)],
                    aggout.at[c, pl.ds(base, _RS)])
    if with_deg:
        pltpu.sync_copy(degsh.at[pl.ds(base, _RS)],
                        degout.at[c, pl.ds(base, _RS)])


@functools.lru_cache(maxsize=None)
def _make_sc_segsum(e_pad, with_deg):
    ew = e_pad // _NW
    n_chunks = ew // _K
    mesh = plsc.VectorSubcoreMesh(core_axis_name="c", subcore_axis_name="s")
    out_type = [jax.ShapeDtypeStruct((_NC, _NPAD, _D), jnp.float32)]
    scratch = [
        pltpu.VMEM((_B, _K), jnp.int32),
        pltpu.VMEM((_B, _K), jnp.int32),
    ]
    scratch += [pltpu.VMEM((_K, _D), jnp.float32)] * _NBUF
    assert 16 * (_NBUF * _K * _D + 2 * _B * _K + _K + _RS) \
        + _NPAD * _D + _NPAD <= 2097151
    if with_deg:
        out_type.append(jax.ShapeDtypeStruct((_NC, _NPAD), jnp.float32))
        scratch += [
            pltpu.VMEM((_K,), jnp.float32),
            pltpu.VMEM((_RS,), jnp.float32),
        ]
    scratch.append(pltpu.VMEM_SHARED((_NPAD, _D), jnp.float32))
    if with_deg:
        scratch.append(pltpu.VMEM_SHARED((_NPAD,), jnp.float32))
    scratch += [pltpu.SemaphoreType.DMA] * (_NBUF * (3 if with_deg else 2))
    return pl.kernel(
        functools.partial(_sc_body, with_deg, n_chunks, ew),
        out_type=tuple(out_type),
        mesh=mesh,
        scratch_types=tuple(scratch),
        name=f"sage_sc_segsum{'_deg' if with_deg else ''}",
    )


def _mm_body(x_r, wl_r, wr_r, bl_r, y_r, z_r):
    h = x_r[...]
    y_r[...] = jnp.dot(h, wl_r[...], precision=_HIGH,
                       preferred_element_type=jnp.float32)
    z_r[...] = jnp.dot(h, wr_r[...], precision=_HIGH,
                       preferred_element_type=jnp.float32) + bl_r[...]


def _combine_mm_body(p_r, d0_r, d1_r, z_r, wl_r, wr_r, bl_r,
                     h_r, y_r, z2_r):
    deg = jnp.maximum(d0_r[...] + d1_r[...], 1.0)
    h = jnp.maximum((p_r[0] + p_r[1]) / deg + z_r[...], 0.0)
    h_r[...] = h
    y_r[...] = jnp.dot(h, wl_r[...], precision=_HIGH,
                       preferred_element_type=jnp.float32)
    z2_r[...] = jnp.dot(h, wr_r[...], precision=_HIGH,
                        preferred_element_type=jnp.float32) + bl_r[...]


def _combine_body(p_r, d0_r, d1_r, z_r, h_r):
    deg = jnp.maximum(d0_r[...] + d1_r[...], 1.0)
    h_r[...] = jnp.maximum((p_r[0] + p_r[1]) / deg + z_r[...], 0.0)


_row_spec = pl.BlockSpec((_R, _D), lambda i: (i, 0))
_w_spec = pl.BlockSpec((_D, _D), lambda i: (0, 0))
_b_spec = pl.BlockSpec((1, _D), lambda i: (0, 0))
_deg_spec = pl.BlockSpec((_R, 1), lambda i: (i, 0))
_p_spec = pl.BlockSpec((_NC, _R, _D), lambda i: (0, i, 0))
_nd = jax.ShapeDtypeStruct((_N, _D), jnp.float32)

_mm = pl.pallas_call(
    _mm_body,
    grid=(_N // _R,),
    in_specs=[_row_spec, _w_spec, _w_spec, _b_spec],
    out_specs=[_row_spec, _row_spec],
    out_shape=[_nd, _nd],
)

_combine_mm = pl.pallas_call(
    _combine_mm_body,
    grid=(_N // _R,),
    in_specs=[_p_spec, _deg_spec, _deg_spec, _row_spec,
              _w_spec, _w_spec, _b_spec],
    out_specs=[_row_spec, _row_spec, _row_spec],
    out_shape=[_nd, _nd, _nd],
)

_combine = pl.pallas_call(
    _combine_body,
    grid=(_N // _R,),
    in_specs=[_p_spec, _deg_spec, _deg_spec, _row_spec],
    out_specs=_row_spec,
    out_shape=_nd,
)


def kernel(x, edge_index, Wl0, bl0, Wr0, Wl1, bl1, Wr1, Wl2, bl2, Wr2):
    src = edge_index[0]
    dst = edge_index[1]
    e = src.shape[0]
    span = _NW * _K * _B
    e_pad = -(-e // span) * span
    pad = e_pad - e
    if pad:
        ar = jnp.arange(pad, dtype=jnp.int32)
        src = jnp.concatenate([src, ar % _N])
        dst = jnp.concatenate([dst, _N + ar % (_NPAD - _N)])
    src = src.reshape(-1, _K)
    dst = dst.reshape(-1, _K)

    sc_deg = _make_sc_segsum(e_pad, True)
    sc = _make_sc_segsum(e_pad, False)

    y0, z0 = _mm(x, Wl0, Wr0, bl0.reshape(1, _D))
    p, pdeg = sc_deg(y0, src, dst)
    d0 = pdeg[0, :_N].reshape(_N, 1)
    d1 = pdeg[1, :_N].reshape(_N, 1)
    h1, y1, z1 = _combine_mm(p, d0, d1, z0, Wl1, Wr1, bl1.reshape(1, _D))
    (q,) = sc(y1, src, dst)
    h2, y2, z2 = _combine_mm(q, d0, d1, z1, Wl2, Wr2, bl2.reshape(1, _D))
    (r,) = sc(y2, src, dst)
    h3 = _combine(r, d0, d1, z2)
    return (h1, h2, h3)


# speedup vs baseline: 1.1282x; 1.0051x over previous
import functools

import jax
import jax.numpy as jnp
from jax import lax
from jax.experimental import pallas as pl
from jax.experimental.pallas import tpu as pltpu
from jax.experimental.pallas import tpu_sc as plsc

_N = 10000
_D = 128
_NC = 2
_NS = 16
_NW = _NC * _NS
_K = 64
_NPAD = 10240
_RS = _NPAD // _NS
_R = 5000

_HIGH = jax.lax.Precision.DEFAULT


_B = 32
_NBUF = 4
_DEPTH = 3


def _sc_body(with_deg, n_chunks, ew, *refs):
    if with_deg:
        (y, srcr, dstr, aggout, degout, src_blk, dst_blk) = refs[:7]
        bufs = refs[7:7 + _NBUF]
        (ones, degtmp, aggsh, degsh) = refs[7 + _NBUF:11 + _NBUF]
        sems = refs[11 + _NBUF:]
    else:
        (y, srcr, dstr, aggout, src_blk, dst_blk) = refs[:6]
        bufs = refs[6:6 + _NBUF]
        aggsh = refs[6 + _NBUF]
        sems = refs[7 + _NBUF:]
    gsems = sems[:_NBUF]
    ssems = sems[_NBUF:2 * _NBUF]
    dsems = sems[2 * _NBUF:]
    rows = bufs[0]

    c = lax.axis_index("c")
    s = lax.axis_index("s")
    w = c * _NS + s

    zeros16 = jnp.zeros((16,), jnp.float32)

    def zrow(i, carry):
        for j in range(_D // 16):
            rows[i, pl.ds(j * 16, 16)] = zeros16
        return carry

    lax.fori_loop(0, _K, zrow, 0)
    base = s * _RS
    for j in range(_RS // _K):
        pltpu.sync_copy(rows, aggsh.at[pl.ds(base + j * _K, _K)])

    if with_deg:
        def zdeg(i, carry):
            degtmp[pl.ds(i * 16, 16)] = zeros16
            return carry

        lax.fori_loop(0, _RS // 16, zdeg, 0)
        pltpu.sync_copy(degtmp, degsh.at[pl.ds(base, _RS)])

        def fill_ones(i, carry):
            ones[pl.ds(i * 16, 16)] = jnp.ones((16,), jnp.float32)
            return carry

        lax.fori_loop(0, _K // 16, fill_ones, 0)

    plsc.subcore_barrier()

    wrow = w * (ew // _K)

    def blk(b, carry):
        r0 = wrow + b * _B
        pltpu.sync_copy(srcr.at[pl.ds(r0, _B)], src_blk)
        pltpu.sync_copy(dstr.at[pl.ds(r0, _B)], dst_blk)
        gd = [None] * _B
        sd = [None] * _B
        dd = [None] * _B

        def scatter(i):
            gd[i].wait()
            sd[i] = pltpu.async_copy(bufs[i % _NBUF],
                                     aggsh.at[dst_blk.at[i]],
                                     ssems[i % _NBUF], add=True)
            if with_deg:
                dd[i] = pltpu.async_copy(ones, degsh.at[dst_blk.at[i]],
                                         dsems[i % _NBUF], add=True)

        for j in range(_B):
            if j >= _NBUF:
                sd[j - _NBUF].wait()
                if with_deg:
                    dd[j - _NBUF].wait()
            gd[j] = pltpu.async_copy(y.at[src_blk.at[j]],
                                     bufs[j % _NBUF], gsems[j % _NBUF])
            if j >= _DEPTH:
                scatter(j - _DEPTH)
        for i in range(_B - _DEPTH, _B):
            scatter(i)
        for i in range(_B - _NBUF, _B):
            sd[i].wait()
            if with_deg:
                dd[i].wait()
        return carry

    lax.fori_loop(0, n_chunks // _B, blk, 0)

    plsc.subcore_barrier()

    pltpu.sync_copy(aggsh.at[pl.ds(base, _RS)],
                    aggout.at[c, pl.ds(base, _RS)])
    if with_deg:
        pltpu.sync_copy(degsh.at[pl.ds(base, _RS)],
                        degout.at[c, pl.ds(base, _RS)])


@functools.lru_cache(maxsize=None)
def _make_sc_segsum(e_pad, with_deg):
    ew = e_pad // _NW
    n_chunks = ew // _K
    mesh = plsc.VectorSubcoreMesh(core_axis_name="c", subcore_axis_name="s")
    out_type = [jax.ShapeDtypeStruct((_NC, _NPAD, _D), jnp.float32)]
    scratch = [
        pltpu.VMEM((_B, _K), jnp.int32),
        pltpu.VMEM((_B, _K), jnp.int32),
    ]
    scratch += [pltpu.VMEM((_K, _D), jnp.float32)] * _NBUF
    assert 16 * (_NBUF * _K * _D + 2 * _B * _K + _K + _RS) \
        + _NPAD * _D + _NPAD <= 2097151
    if with_deg:
        out_type.append(jax.ShapeDtypeStruct((_NC, _NPAD), jnp.float32))
        scratch += [
            pltpu.VMEM((_K,), jnp.float32),
            pltpu.VMEM((_RS,), jnp.float32),
        ]
    scratch.append(pltpu.VMEM_SHARED((_NPAD, _D), jnp.float32))
    if with_deg:
        scratch.append(pltpu.VMEM_SHARED((_NPAD,), jnp.float32))
    scratch += [pltpu.SemaphoreType.DMA] * (_NBUF * (3 if with_deg else 2))
    return pl.kernel(
        functools.partial(_sc_body, with_deg, n_chunks, ew),
        out_type=tuple(out_type),
        mesh=mesh,
        scratch_types=tuple(scratch),
        name=f"sage_sc_segsum{'_deg' if with_deg else ''}",
    )


def _mm_body(x_r, wl_r, wr_r, bl_r, y_r, z_r):
    h = x_r[...]
    y_r[...] = jnp.dot(h, wl_r[...], precision=_HIGH,
                       preferred_element_type=jnp.float32)
    z_r[...] = jnp.dot(h, wr_r[...], precision=_HIGH,
                       preferred_element_type=jnp.float32) + bl_r[...]


def _combine_mm_body(p_r, d0_r, d1_r, z_r, wl_r, wr_r, bl_r,
                     h_r, y_r, z2_r):
    deg = jnp.maximum(d0_r[...] + d1_r[...], 1.0)
    h = jnp.maximum((p_r[0] + p_r[1]) / deg + z_r[...], 0.0)
    h_r[...] = h
    y_r[...] = jnp.dot(h, wl_r[...], precision=_HIGH,
                       preferred_element_type=jnp.float32)
    z2_r[...] = jnp.dot(h, wr_r[...], precision=_HIGH,
                        preferred_element_type=jnp.float32) + bl_r[...]


def _combine_body(p_r, d0_r, d1_r, z_r, h_r):
    deg = jnp.maximum(d0_r[...] + d1_r[...], 1.0)
    h_r[...] = jnp.maximum((p_r[0] + p_r[1]) / deg + z_r[...], 0.0)


_row_spec = pl.BlockSpec((_R, _D), lambda i: (i, 0))
_w_spec = pl.BlockSpec((_D, _D), lambda i: (0, 0))
_b_spec = pl.BlockSpec((1, _D), lambda i: (0, 0))
_deg_spec = pl.BlockSpec((_R, 1), lambda i: (i, 0))
_p_spec = pl.BlockSpec((_NC, _R, _D), lambda i: (0, i, 0))
_nd = jax.ShapeDtypeStruct((_N, _D), jnp.float32)

_mm = pl.pallas_call(
    _mm_body,
    grid=(_N // _R,),
    in_specs=[_row_spec, _w_spec, _w_spec, _b_spec],
    out_specs=[_row_spec, _row_spec],
    out_shape=[_nd, _nd],
)

_combine_mm = pl.pallas_call(
    _combine_mm_body,
    grid=(_N // _R,),
    in_specs=[_p_spec, _deg_spec, _deg_spec, _row_spec,
              _w_spec, _w_spec, _b_spec],
    out_specs=[_row_spec, _row_spec, _row_spec],
    out_shape=[_nd, _nd, _nd],
)

_combine = pl.pallas_call(
    _combine_body,
    grid=(_N // _R,),
    in_specs=[_p_spec, _deg_spec, _deg_spec, _row_spec],
    out_specs=_row_spec,
    out_shape=_nd,
)


def kernel(x, edge_index, Wl0, bl0, Wr0, Wl1, bl1, Wr1, Wl2, bl2, Wr2):
    src = edge_index[0]
    dst = edge_index[1]
    e = src.shape[0]
    span = _NW * _K * _B
    e_pad = -(-e // span) * span
    pad = e_pad - e
    if pad:
        ar = jnp.arange(pad, dtype=jnp.int32)
        src = jnp.concatenate([src, ar % _N])
        dst = jnp.concatenate([dst, _N + ar % (_NPAD - _N)])
    src = src.reshape(-1, _K)
    dst = dst.reshape(-1, _K)

    sc_deg = _make_sc_segsum(e_pad, True)
    sc = _make_sc_segsum(e_pad, False)

    y0, z0 = _mm(x, Wl0, Wr0, bl0.reshape(1, _D))
    p, pdeg = sc_deg(y0, src, dst)
    d0 = pdeg[0, :_N].reshape(_N, 1)
    d1 = pdeg[1, :_N].reshape(_N, 1)
    h1, y1, z1 = _combine_mm(p, d0, d1, z0, Wl1, Wr1, bl1.reshape(1, _D))
    (q,) = sc(y1, src, dst)
    h2, y2, z2 = _combine_mm(q, d0, d1, z1, Wl2, Wr2, bl2.reshape(1, _D))
    (r,) = sc(y2, src, dst)
    h3 = _combine(r, d0, d1, z2)
    return (h1, h2, h3)
